# baseline (speedup 1.0000x reference)
import functools

import jax
import jax.numpy as jnp
from jax import lax
from jax.experimental import pallas as pl
from jax.experimental.pallas import tpu as pltpu
from jax.experimental.pallas import tpu_sc as plsc

N_POS = 8192
D_MODEL = 1024
N_CORES = 2
N_SUBCORES = 16
N_WORKERS = N_CORES * N_SUBCORES
ROWS_PER_W = N_POS // N_WORKERS

CHUNK = 40
NBUF = 3
_full = ROWS_PER_W // CHUNK
_tail = ROWS_PER_W - _full * CHUNK
CHUNK_SIZES = [CHUNK] * _full + ([_tail] if _tail else [])
CHUNK_OFFS = [sum(CHUNK_SIZES[:i]) for i in range(len(CHUNK_SIZES))]
N_CHUNKS = len(CHUNK_SIZES)


@functools.partial(
    pl.kernel,
    mesh=plsc.VectorSubcoreMesh(core_axis_name="c", subcore_axis_name="s"),
    out_type=jax.ShapeDtypeStruct((N_POS, D_MODEL), jnp.float32),
    scratch_types=[
        pltpu.VMEM((CHUNK, D_MODEL), jnp.float32),
        pltpu.VMEM((CHUNK, D_MODEL), jnp.float32),
        pltpu.VMEM((CHUNK, D_MODEL), jnp.float32),
        pltpu.SemaphoreType.DMA,
        pltpu.SemaphoreType.DMA,
        pltpu.SemaphoreType.DMA,
        pltpu.SemaphoreType.DMA,
        pltpu.SemaphoreType.DMA,
        pltpu.SemaphoreType.DMA,
    ],
)
def _sc_copy(table_hbm, out_hbm, buf0, buf1, buf2,
             rsem0, rsem1, rsem2, wsem0, wsem1, wsem2):
    wid = lax.axis_index("s") * N_CORES + lax.axis_index("c")
    base = wid * ROWS_PER_W
    bufs = (buf0, buf1, buf2)
    rsems = (rsem0, rsem1, rsem2)
    wsems = (wsem0, wsem1, wsem2)

    def rd(i):
        b, n = i % NBUF, CHUNK_SIZES[i]
        return pltpu.make_async_copy(
            table_hbm.at[pl.ds(base + CHUNK_OFFS[i], n)],
            bufs[b].at[pl.ds(0, n)], rsems[b])

    def wr(i):
        b, n = i % NBUF, CHUNK_SIZES[i]
        return pltpu.make_async_copy(
            bufs[b].at[pl.ds(0, n)],
            out_hbm.at[pl.ds(base + CHUNK_OFFS[i], n)], wsems[b])

    for i in range(min(NBUF, N_CHUNKS)):
        rd(i).start()
    for i in range(N_CHUNKS):
        rd(i).wait()
        wr(i).start()
        if i >= 1 and i + 2 < N_CHUNKS:
            wr(i - 1).wait()
            rd(i + 2).start()
    for i in range(max(0, N_CHUNKS - NBUF), N_CHUNKS):
        wr(i).wait()


def kernel(x, positional_embedding):
    del x
    return _sc_copy(positional_embedding)


# speedup vs baseline: 1.0270x; 1.0270x over previous
import functools

import jax
import jax.numpy as jnp
from jax import lax
from jax.experimental import pallas as pl
from jax.experimental.pallas import tpu as pltpu
from jax.experimental.pallas import tpu_sc as plsc

N_POS = 8192
D_MODEL = 1024
N_CORES = 2
N_SUBCORES = 16
N_WORKERS = N_CORES * N_SUBCORES
ROWS_PER_W = N_POS // N_WORKERS

CHUNK = 24
NBUF = 5
_full = ROWS_PER_W // CHUNK
_tail = ROWS_PER_W - _full * CHUNK
CHUNK_SIZES = [CHUNK] * _full + ([_tail] if _tail else [])
CHUNK_OFFS = [sum(CHUNK_SIZES[:i]) for i in range(len(CHUNK_SIZES))]
N_CHUNKS = len(CHUNK_SIZES)


@functools.partial(
    pl.kernel,
    mesh=plsc.VectorSubcoreMesh(core_axis_name="c", subcore_axis_name="s"),
    out_type=jax.ShapeDtypeStruct((N_POS, D_MODEL), jnp.float32),
    scratch_types=(
        [pltpu.VMEM((CHUNK, D_MODEL), jnp.float32)] * NBUF
        + [pltpu.SemaphoreType.DMA] * (2 * NBUF)
    ),
)
def _sc_copy(table_hbm, out_hbm, *scratch):
    wid = lax.axis_index("s") * N_CORES + lax.axis_index("c")
    base = wid * ROWS_PER_W
    bufs = scratch[:NBUF]
    rsems = scratch[NBUF:2 * NBUF]
    wsems = scratch[2 * NBUF:]

    def rd(i):
        b, n = i % NBUF, CHUNK_SIZES[i]
        return pltpu.make_async_copy(
            table_hbm.at[pl.ds(base + CHUNK_OFFS[i], n)],
            bufs[b].at[pl.ds(0, n)], rsems[b])

    def wr(i):
        b, n = i % NBUF, CHUNK_SIZES[i]
        return pltpu.make_async_copy(
            bufs[b].at[pl.ds(0, n)],
            out_hbm.at[pl.ds(base + CHUNK_OFFS[i], n)], wsems[b])

    for i in range(min(NBUF, N_CHUNKS)):
        rd(i).start()
    for i in range(N_CHUNKS):
        rd(i).wait()
        wr(i).start()
        if i >= 1 and i + NBUF - 1 < N_CHUNKS:
            wr(i - 1).wait()
            rd(i + NBUF - 1).start()
    for i in range(max(0, N_CHUNKS - NBUF), N_CHUNKS):
        wr(i).wait()


def kernel(x, positional_embedding):
    del x
    return _sc_copy(positional_embedding)
